# Initial kernel scaffold; baseline (speedup 1.0000x reference)
import jax
import jax.numpy as jnp
from jax.experimental import pallas as pl


def kernel(x, edge_index, W1, b1, W2, b2, Wg, a_src, a_dst, alphas, Wf, bf):
    raise NotImplementedError("write your pallas kernel here")


# speedup vs baseline: 13.5122x; 13.5122x over previous
import functools

import jax
import jax.numpy as jnp
from jax import lax
from jax.experimental import pallas as pl
from jax.experimental.pallas import tpu as pltpu
from jax.experimental.pallas import tpu_sc as plsc

_N = 10000
_E = 320000
_D = 128
_L = 32
_OUT = 128

_NC = 2
_NS = 16
_NPAD = 10240
_STRIPE = _NPAD // _NS
_EPT = _E // (_NC * _NS)
_CH = 2000
_NCHUNK = _EPT // _CH
_B = 80
_R = _CH // _B


def _tc_pre_body(x_ref, w1_ref, b1_ref, w2_ref, b2_ref, wg_ref, a2_ref,
                 regre_ref, h_ref, sd_ref):
    x = x_ref[...]
    t = jnp.maximum(jnp.dot(x, w1_ref[...],
                            preferred_element_type=jnp.float32)
                    + b1_ref[...][None, :], 0.0)
    regre_ref[...] = jnp.dot(t, w2_ref[...],
                             preferred_element_type=jnp.float32) \
        + b2_ref[...][None, :]
    h = jnp.dot(x, wg_ref[...], preferred_element_type=jnp.float32)
    h_ref[...] = h
    sd_ref[...] = jnp.dot(h, a2_ref[...], preferred_element_type=jnp.float32)


def _tc_pre(x, W1, b1, W2, b2, Wg, a2):
    return pl.pallas_call(
        _tc_pre_body,
        out_shape=(
            jax.ShapeDtypeStruct((_N, _L), jnp.float32),
            jax.ShapeDtypeStruct((_N, _L), jnp.float32),
            jax.ShapeDtypeStruct((_N, 2), jnp.float32),
        ),
    )(x, W1, b1, W2, b2, Wg, a2)


def _sc_edge_body(src3, dst3, s_hbm, d_hbm, h_hbm,
                  acc_out, den_out,
                  s_loc, d_loc, src_c, dst_c, ex_c, rows, zbuf,
                  acc_sh, den_sh, sem):
    cid = lax.axis_index("c")
    sid = lax.axis_index("s")
    tid = cid * _NS + sid

    pltpu.sync_copy(s_hbm, s_loc)
    pltpu.sync_copy(d_hbm, d_loc)

    zero16 = jnp.zeros((16,), jnp.float32)

    def _zb(i, c):
        zbuf[pl.ds(i * 16, 16)] = zero16
        return c
    lax.fori_loop(0, _STRIPE // 16, _zb, 0)

    def _zr(i, c):
        rows[i, pl.ds(0, 16)] = zero16
        rows[i, pl.ds(16, 16)] = zero16
        return c
    lax.fori_loop(0, _STRIPE, _zr, 0)

    pltpu.sync_copy(rows.at[pl.ds(0, _STRIPE), :],
                    acc_sh.at[pl.ds(sid * _STRIPE, _STRIPE), :])
    pltpu.sync_copy(zbuf, den_sh.at[pl.ds(sid * _STRIPE, _STRIPE)])
    plsc.subcore_barrier()

    def _chunk(ch, carry):
        blk = tid * _NCHUNK + ch
        pltpu.sync_copy(src3.at[blk], src_c)
        pltpu.sync_copy(dst3.at[blk], dst_c)

        descs = [
            pltpu.async_copy(h_hbm.at[src_c.at[r]],
                             rows.at[pl.ds(r * _B, _B), :], sem)
            for r in range(_R)
        ]

        def _ex_row(r, c):
            for k in range(_B // 16):
                sv = src_c[r, pl.ds(k * 16, 16)]
                dv = dst_c[r, pl.ds(k * 16, 16)]
                e = plsc.load_gather(s_loc, [sv]) \
                    + plsc.load_gather(d_loc, [dv])
                e = jnp.where(e >= 0.0, e, 0.2 * e)
                ex_c[r, pl.ds(k * 16, 16)] = jnp.exp(e)
            return c
        lax.fori_loop(0, _R, _ex_row, 0)

        for dsc in descs:
            dsc.wait()

        iota16 = lax.iota(jnp.int32, 16)

        def _scale(g, c):
            ridx = iota16 + g * 16
            ex_v = ex_c[g // 5, pl.ds((g % 5) * 16, 16)]
            for col in range(_L):
                cidx = jnp.full((16,), col, jnp.int32)
                v = plsc.load_gather(rows, [ridx, cidx])
                plsc.store_scatter(rows, [ridx, cidx], v * ex_v)
            return c
        lax.fori_loop(0, _CH // 16, _scale, 0)

        def _scat(r, c):
            pltpu.sync_copy(rows.at[pl.ds(r * _B, _B), :],
                            acc_sh.at[dst_c.at[r]], add=True)
            pltpu.sync_copy(ex_c.at[r], den_sh.at[dst_c.at[r]], add=True)
            return c
        lax.fori_loop(0, _R, _scat, 0)
        return carry

    lax.fori_loop(0, _NCHUNK, _chunk, 0)
    plsc.subcore_barrier()

    pltpu.sync_copy(acc_sh.at[pl.ds(sid * _STRIPE, _STRIPE), :],
                    rows.at[pl.ds(0, _STRIPE), :])
    pltpu.sync_copy(rows.at[pl.ds(0, _STRIPE), :],
                    acc_out.at[cid, pl.ds(sid * _STRIPE, _STRIPE), :])
    pltpu.sync_copy(den_sh.at[pl.ds(sid * _STRIPE, _STRIPE)], zbuf)
    pltpu.sync_copy(zbuf, den_out.at[cid, pl.ds(sid * _STRIPE, _STRIPE)])


def _sc_edge(src3, dst3, s, d, h):
    mesh = plsc.VectorSubcoreMesh(core_axis_name="c", subcore_axis_name="s",
                                  num_cores=_NC, num_subcores=_NS)
    f = pl.kernel(
        _sc_edge_body,
        out_type=(
            jax.ShapeDtypeStruct((_NC, _NPAD, _L), jnp.float32),
            jax.ShapeDtypeStruct((_NC, _NPAD), jnp.float32),
        ),
        mesh=mesh,
        compiler_params=pltpu.CompilerParams(needs_layout_passes=False,
                                             use_tc_tiling_on_sc=False),
        scratch_types=[
            pltpu.VMEM((_N,), jnp.float32),
            pltpu.VMEM((_N,), jnp.float32),
            pltpu.VMEM((_R, _B), jnp.int32),
            pltpu.VMEM((_R, _B), jnp.int32),
            pltpu.VMEM((_R, _B), jnp.float32),
            pltpu.VMEM((_CH, _L), jnp.float32),
            pltpu.VMEM((_STRIPE,), jnp.float32),
            pltpu.VMEM_SHARED((_NPAD, _L), jnp.float32),
            pltpu.VMEM_SHARED((_NPAD,), jnp.float32),
            pltpu.SemaphoreType.DMA,
        ],
    )
    return f(src3, dst3, s, d, h)


def _tc_post_body(regre_ref, acc_ref, den_ref, alphas_ref, wf_ref, bf_ref,
                  out_ref):
    acc = acc_ref[0] + acc_ref[1]
    den = den_ref[...]
    tot = den[:, 0:1] + den[:, 1:2]
    interp = acc / (tot + 1e-16)
    al = alphas_ref[...][None, :]
    mix = regre_ref[...] * al + interp * (1.0 - al)
    out_ref[...] = jnp.maximum(
        jnp.dot(mix, wf_ref[...], preferred_element_type=jnp.float32)
        + bf_ref[...][None, :], 0.0)


def _tc_post(regre, acc, den_t, alphas, Wf, bf):
    return pl.pallas_call(
        _tc_post_body,
        out_shape=jax.ShapeDtypeStruct((_N, _OUT), jnp.float32),
    )(regre, acc, den_t, alphas, Wf, bf)


def kernel(x, edge_index, W1, b1, W2, b2, Wg, a_src, a_dst, alphas, Wf, bf):
    a2 = jnp.stack([a_src, a_dst], axis=1)
    regre, h, sd = _tc_pre(x, W1, b1, W2, b2, Wg, a2)
    s = sd[:, 0]
    d = sd[:, 1]
    src3 = edge_index[0].reshape(_NC * _NS * _NCHUNK, _R, _B)
    dst3 = edge_index[1].reshape(_NC * _NS * _NCHUNK, _R, _B)
    acc, den = _sc_edge(src3, dst3, s, d, h)
    acc = acc[:, :_N, :]
    den_t = den[:, :_N].T
    return _tc_post(regre, acc, den_t, alphas, Wf, bf)


# speedup vs baseline: 13.9638x; 1.0334x over previous
import functools

import jax
import jax.numpy as jnp
from jax import lax
from jax.experimental import pallas as pl
from jax.experimental.pallas import tpu as pltpu
from jax.experimental.pallas import tpu_sc as plsc

_N = 10000
_E = 320000
_D = 128
_L = 32
_OUT = 128

_NC = 2
_NS = 16
_NPAD = 10240
_STRIPE = _NPAD // _NS
_EPT = _E // (_NC * _NS)
_CH = 2000
_NCHUNK = _EPT // _CH
_B = 80
_R = _CH // _B


def _tc_pre_body(x_ref, w1_ref, b1_ref, w2_ref, b2_ref, wg_ref, a2_ref,
                 regre_ref, h_ref, sd_ref):
    x = x_ref[...]
    t = jnp.maximum(jnp.dot(x, w1_ref[...],
                            preferred_element_type=jnp.float32)
                    + b1_ref[...][None, :], 0.0)
    regre_ref[...] = jnp.dot(t, w2_ref[...],
                             preferred_element_type=jnp.float32) \
        + b2_ref[...][None, :]
    h = jnp.dot(x, wg_ref[...], preferred_element_type=jnp.float32)
    h_ref[...] = h
    sd_ref[...] = jnp.dot(h, a2_ref[...], preferred_element_type=jnp.float32)


def _tc_pre(x, W1, b1, W2, b2, Wg, a2):
    return pl.pallas_call(
        _tc_pre_body,
        out_shape=(
            jax.ShapeDtypeStruct((_N, _L), jnp.float32),
            jax.ShapeDtypeStruct((_N, _L), jnp.float32),
            jax.ShapeDtypeStruct((_N, 2), jnp.float32),
        ),
    )(x, W1, b1, W2, b2, Wg, a2)


def _sc_edge_body(src3, dst3, s_hbm, d_hbm, h_hbm,
                  acc_out, den_out,
                  s_loc, d_loc, src_c, dst_c, ex_c, rows, zbuf,
                  acc_sh, den_sh, sem, sem2):
    cid = lax.axis_index("c")
    sid = lax.axis_index("s")
    tid = cid * _NS + sid

    pltpu.sync_copy(s_hbm, s_loc)
    pltpu.sync_copy(d_hbm, d_loc)

    zero16 = jnp.zeros((16,), jnp.float32)

    def _zb(i, c):
        zbuf[pl.ds(i * 16, 16)] = zero16
        return c
    lax.fori_loop(0, _STRIPE // 16, _zb, 0)

    def _zr(i, c):
        rows[i, pl.ds(0, 16)] = zero16
        rows[i, pl.ds(16, 16)] = zero16
        return c
    lax.fori_loop(0, _STRIPE, _zr, 0)

    pltpu.sync_copy(rows.at[pl.ds(0, _STRIPE), :],
                    acc_sh.at[pl.ds(sid * _STRIPE, _STRIPE), :])
    pltpu.sync_copy(zbuf, den_sh.at[pl.ds(sid * _STRIPE, _STRIPE)])
    plsc.subcore_barrier()

    def _chunk(ch, carry):
        blk = tid * _NCHUNK + ch
        di = [pltpu.async_copy(src3.at[blk], src_c, sem),
              pltpu.async_copy(dst3.at[blk], dst_c, sem)]
        for dsc in di:
            dsc.wait()

        descs = [
            pltpu.async_copy(h_hbm.at[src_c.at[r]],
                             rows.at[pl.ds(r * _B, _B), :], sem)
            for r in range(_R)
        ]

        def _ex_row(r, c):
            for k in range(_B // 16):
                sv = src_c[r, pl.ds(k * 16, 16)]
                dv = dst_c[r, pl.ds(k * 16, 16)]
                e = plsc.load_gather(s_loc, [sv]) \
                    + plsc.load_gather(d_loc, [dv])
                e = jnp.where(e >= 0.0, e, 0.2 * e)
                ex_c[r, pl.ds(k * 16, 16)] = jnp.exp(e)
            return c
        lax.fori_loop(0, _R, _ex_row, 0)

        for dsc in descs:
            dsc.wait()

        iota16 = lax.iota(jnp.int32, 16)

        def _scale(g, c):
            ridx = iota16 + g * 16
            ex_v = ex_c[g // 5, pl.ds((g % 5) * 16, 16)]
            for col in range(_L):
                cidx = jnp.full((16,), col, jnp.int32)
                v = plsc.load_gather(rows, [ridx, cidx])
                plsc.store_scatter(rows, [ridx, cidx], v * ex_v)
            return c
        lax.fori_loop(0, _CH // 16, _scale, 0)

        ds = []
        for r in range(_R):
            ds.append(pltpu.async_copy(rows.at[pl.ds(r * _B, _B), :],
                                       acc_sh.at[dst_c.at[r]], sem2,
                                       add=True))
            ds.append(pltpu.async_copy(ex_c.at[r], den_sh.at[dst_c.at[r]],
                                       sem2, add=True))
        for dsc in ds:
            dsc.wait()
        return carry

    lax.fori_loop(0, _NCHUNK, _chunk, 0)
    plsc.subcore_barrier()

    pltpu.sync_copy(acc_sh.at[pl.ds(sid * _STRIPE, _STRIPE), :],
                    rows.at[pl.ds(0, _STRIPE), :])
    pltpu.sync_copy(rows.at[pl.ds(0, _STRIPE), :],
                    acc_out.at[cid, pl.ds(sid * _STRIPE, _STRIPE), :])
    pltpu.sync_copy(den_sh.at[pl.ds(sid * _STRIPE, _STRIPE)], zbuf)
    pltpu.sync_copy(zbuf, den_out.at[cid, pl.ds(sid * _STRIPE, _STRIPE)])


def _sc_edge(src3, dst3, s, d, h):
    mesh = plsc.VectorSubcoreMesh(core_axis_name="c", subcore_axis_name="s",
                                  num_cores=_NC, num_subcores=_NS)
    f = pl.kernel(
        _sc_edge_body,
        out_type=(
            jax.ShapeDtypeStruct((_NC, _NPAD, _L), jnp.float32),
            jax.ShapeDtypeStruct((_NC, _NPAD), jnp.float32),
        ),
        mesh=mesh,
        compiler_params=pltpu.CompilerParams(needs_layout_passes=False,
                                             use_tc_tiling_on_sc=False),
        scratch_types=[
            pltpu.VMEM((_N,), jnp.float32),
            pltpu.VMEM((_N,), jnp.float32),
            pltpu.VMEM((_R, _B), jnp.int32),
            pltpu.VMEM((_R, _B), jnp.int32),
            pltpu.VMEM((_R, _B), jnp.float32),
            pltpu.VMEM((_CH, _L), jnp.float32),
            pltpu.VMEM((_STRIPE,), jnp.float32),
            pltpu.VMEM_SHARED((_NPAD, _L), jnp.float32),
            pltpu.VMEM_SHARED((_NPAD,), jnp.float32),
            pltpu.SemaphoreType.DMA,
            pltpu.SemaphoreType.DMA,
        ],
    )
    return f(src3, dst3, s, d, h)


def _tc_post_body(regre_ref, acc_ref, den_ref, alphas_ref, wf_ref, bf_ref,
                  out_ref):
    acc = acc_ref[0] + acc_ref[1]
    den = den_ref[...]
    tot = den[:, 0:1] + den[:, 1:2]
    interp = acc / (tot + 1e-16)
    al = alphas_ref[...][None, :]
    mix = regre_ref[...] * al + interp * (1.0 - al)
    out_ref[...] = jnp.maximum(
        jnp.dot(mix, wf_ref[...], preferred_element_type=jnp.float32)
        + bf_ref[...][None, :], 0.0)


def _tc_post(regre, acc, den_t, alphas, Wf, bf):
    return pl.pallas_call(
        _tc_post_body,
        out_shape=jax.ShapeDtypeStruct((_N, _OUT), jnp.float32),
    )(regre, acc, den_t, alphas, Wf, bf)


def kernel(x, edge_index, W1, b1, W2, b2, Wg, a_src, a_dst, alphas, Wf, bf):
    a2 = jnp.stack([a_src, a_dst], axis=1)
    regre, h, sd = _tc_pre(x, W1, b1, W2, b2, Wg, a2)
    s = sd[:, 0]
    d = sd[:, 1]
    src3 = edge_index[0].reshape(_NC * _NS * _NCHUNK, _R, _B)
    dst3 = edge_index[1].reshape(_NC * _NS * _NCHUNK, _R, _B)
    acc, den = _sc_edge(src3, dst3, s, d, h)
    acc = acc[:, :_N, :]
    den_t = den[:, :_N].T
    return _tc_post(regre, acc, den_t, alphas, Wf, bf)


# speedup vs baseline: 14.2135x; 1.0179x over previous
import functools

import jax
import jax.numpy as jnp
from jax import lax
from jax.experimental import pallas as pl
from jax.experimental.pallas import tpu as pltpu
from jax.experimental.pallas import tpu_sc as plsc

_N = 10000
_E = 320000
_D = 128
_L = 32
_OUT = 128

_NC = 2
_NS = 16
_NPAD = 10240
_STRIPE = _NPAD // _NS
_EPT = _E // (_NC * _NS)
_CH = 2000
_NCHUNK = _EPT // _CH
_B = 80
_R = _CH // _B

_ABL_GATHER = False
_ABL_SCALE = True
_ABL_SCATTER = True


def _tc_pre_body(x_ref, w1_ref, b1_ref, w2_ref, b2_ref, wg_ref, a2_ref,
                 regre_ref, h_ref, sd_ref):
    x = x_ref[...]
    t = jnp.maximum(jnp.dot(x, w1_ref[...],
                            preferred_element_type=jnp.float32)
                    + b1_ref[...][None, :], 0.0)
    regre_ref[...] = jnp.dot(t, w2_ref[...],
                             preferred_element_type=jnp.float32) \
        + b2_ref[...][None, :]
    h = jnp.dot(x, wg_ref[...], preferred_element_type=jnp.float32)
    h_ref[...] = h
    sd_ref[...] = jnp.dot(h, a2_ref[...], preferred_element_type=jnp.float32)


def _tc_pre(x, W1, b1, W2, b2, Wg, a2):
    return pl.pallas_call(
        _tc_pre_body,
        out_shape=(
            jax.ShapeDtypeStruct((_N, _L), jnp.float32),
            jax.ShapeDtypeStruct((_N, _L), jnp.float32),
            jax.ShapeDtypeStruct((_N, 2), jnp.float32),
        ),
    )(x, W1, b1, W2, b2, Wg, a2)


def _sc_edge_body(src3, dst3, s_hbm, d_hbm, h_hbm,
                  acc_out, den_out,
                  s_loc, d_loc, src_c, dst_c, ex_c, rows, zbuf,
                  acc_sh, den_sh, sem, sem2):
    cid = lax.axis_index("c")
    sid = lax.axis_index("s")
    tid = cid * _NS + sid

    pltpu.sync_copy(s_hbm, s_loc)
    pltpu.sync_copy(d_hbm, d_loc)

    zero16 = jnp.zeros((16,), jnp.float32)

    def _zb(i, c):
        zbuf[pl.ds(i * 16, 16)] = zero16
        return c
    lax.fori_loop(0, _STRIPE // 16, _zb, 0)

    def _zr(i, c):
        rows[i, pl.ds(0, 16)] = zero16
        rows[i, pl.ds(16, 16)] = zero16
        return c
    lax.fori_loop(0, _STRIPE, _zr, 0)

    pltpu.sync_copy(rows.at[pl.ds(0, _STRIPE), :],
                    acc_sh.at[pl.ds(sid * _STRIPE, _STRIPE), :])
    pltpu.sync_copy(zbuf, den_sh.at[pl.ds(sid * _STRIPE, _STRIPE)])
    plsc.subcore_barrier()

    def _chunk(ch, carry):
        blk = tid * _NCHUNK + ch
        di = [pltpu.async_copy(src3.at[blk], src_c, sem),
              pltpu.async_copy(dst3.at[blk], dst_c, sem)]
        for dsc in di:
            dsc.wait()

        descs = [
            pltpu.async_copy(h_hbm.at[src_c.at[r]],
                             rows.at[pl.ds(r * _B, _B), :], sem)
            for r in range(_R)
        ] if _ABL_GATHER else []

        def _ex_row(r, c):
            for k in range(_B // 16):
                sv = src_c[r, pl.ds(k * 16, 16)]
                dv = dst_c[r, pl.ds(k * 16, 16)]
                e = plsc.load_gather(s_loc, [sv]) \
                    + plsc.load_gather(d_loc, [dv])
                e = jnp.where(e >= 0.0, e, 0.2 * e)
                ex_c[r, pl.ds(k * 16, 16)] = jnp.exp(e)
            return c
        lax.fori_loop(0, _R, _ex_row, 0)

        for dsc in descs:
            dsc.wait()

        iota16 = lax.iota(jnp.int32, 16)

        def _scale(g, c):
            ridx = iota16 + g * 16
            ex_v = ex_c[g // 5, pl.ds((g % 5) * 16, 16)]
            for col in range(_L):
                cidx = jnp.full((16,), col, jnp.int32)
                v = plsc.load_gather(rows, [ridx, cidx])
                plsc.store_scatter(rows, [ridx, cidx], v * ex_v)
            return c
        if _ABL_SCALE:
            lax.fori_loop(0, _CH // 16, _scale, 0)

        ds = []
        for r in range(_R if _ABL_SCATTER else 0):
            ds.append(pltpu.async_copy(rows.at[pl.ds(r * _B, _B), :],
                                       acc_sh.at[dst_c.at[r]], sem2,
                                       add=True))
            ds.append(pltpu.async_copy(ex_c.at[r], den_sh.at[dst_c.at[r]],
                                       sem2, add=True))
        for dsc in ds:
            dsc.wait()
        return carry

    lax.fori_loop(0, _NCHUNK, _chunk, 0)
    plsc.subcore_barrier()

    pltpu.sync_copy(acc_sh.at[pl.ds(sid * _STRIPE, _STRIPE), :],
                    rows.at[pl.ds(0, _STRIPE), :])
    pltpu.sync_copy(rows.at[pl.ds(0, _STRIPE), :],
                    acc_out.at[cid, pl.ds(sid * _STRIPE, _STRIPE), :])
    pltpu.sync_copy(den_sh.at[pl.ds(sid * _STRIPE, _STRIPE)], zbuf)
    pltpu.sync_copy(zbuf, den_out.at[cid, pl.ds(sid * _STRIPE, _STRIPE)])


def _sc_edge(src3, dst3, s, d, h):
    mesh = plsc.VectorSubcoreMesh(core_axis_name="c", subcore_axis_name="s",
                                  num_cores=_NC, num_subcores=_NS)
    f = pl.kernel(
        _sc_edge_body,
        out_type=(
            jax.ShapeDtypeStruct((_NC, _NPAD, _L), jnp.float32),
            jax.ShapeDtypeStruct((_NC, _NPAD), jnp.float32),
        ),
        mesh=mesh,
        compiler_params=pltpu.CompilerParams(needs_layout_passes=False,
                                             use_tc_tiling_on_sc=False),
        scratch_types=[
            pltpu.VMEM((_N,), jnp.float32),
            pltpu.VMEM((_N,), jnp.float32),
            pltpu.VMEM((_R, _B), jnp.int32),
            pltpu.VMEM((_R, _B), jnp.int32),
            pltpu.VMEM((_R, _B), jnp.float32),
            pltpu.VMEM((_CH, _L), jnp.float32),
            pltpu.VMEM((_STRIPE,), jnp.float32),
            pltpu.VMEM_SHARED((_NPAD, _L), jnp.float32),
            pltpu.VMEM_SHARED((_NPAD,), jnp.float32),
            pltpu.SemaphoreType.DMA,
            pltpu.SemaphoreType.DMA,
        ],
    )
    return f(src3, dst3, s, d, h)


def _tc_post_body(regre_ref, acc_ref, den_ref, alphas_ref, wf_ref, bf_ref,
                  out_ref):
    acc = acc_ref[0] + acc_ref[1]
    den = den_ref[...]
    tot = den[:, 0:1] + den[:, 1:2]
    interp = acc / (tot + 1e-16)
    al = alphas_ref[...][None, :]
    mix = regre_ref[...] * al + interp * (1.0 - al)
    out_ref[...] = jnp.maximum(
        jnp.dot(mix, wf_ref[...], preferred_element_type=jnp.float32)
        + bf_ref[...][None, :], 0.0)


def _tc_post(regre, acc, den_t, alphas, Wf, bf):
    return pl.pallas_call(
        _tc_post_body,
        out_shape=jax.ShapeDtypeStruct((_N, _OUT), jnp.float32),
    )(regre, acc, den_t, alphas, Wf, bf)


def kernel(x, edge_index, W1, b1, W2, b2, Wg, a_src, a_dst, alphas, Wf, bf):
    a2 = jnp.stack([a_src, a_dst], axis=1)
    regre, h, sd = _tc_pre(x, W1, b1, W2, b2, Wg, a2)
    s = sd[:, 0]
    d = sd[:, 1]
    src3 = edge_index[0].reshape(_NC * _NS * _NCHUNK, _R, _B)
    dst3 = edge_index[1].reshape(_NC * _NS * _NCHUNK, _R, _B)
    acc, den = _sc_edge(src3, dst3, s, d, h)
    acc = acc[:, :_N, :]
    den_t = den[:, :_N].T
    return _tc_post(regre, acc, den_t, alphas, Wf, bf)
